# baseline (speedup 1.0000x reference)
import functools

import jax
import jax.numpy as jnp
from jax import lax
from jax.experimental import pallas as pl
from jax.experimental.pallas import tpu as pltpu
from jax.experimental.pallas import tpu_sc as plsc

VOCAB = 1000000
DIM = 32
B = 16384
L = 200

VB = 32768
N_VBLK = 31
PV = N_VBLK * VB

NC, NS = 2, 16
NW = NC * NS
N_TOK = B * L
PER_W = N_TOK // NW
CHUNK = 4096
N_CHUNKS = PER_W // CHUNK
SEG = PV // NS


def _project_body(x_ref, w_ref, b_ref, o_ref):
    o_ref[:] = jnp.sum(x_ref[:] * w_ref[:], axis=0) + b_ref[0, 0]


def _project(table_t, w_col, b2):
    return pl.pallas_call(
        _project_body,
        grid=(N_VBLK,),
        in_specs=[
            pl.BlockSpec((DIM, VB), lambda i: (0, i)),
            pl.BlockSpec((DIM, 1), lambda i: (0, 0)),
            pl.BlockSpec((1, 1), lambda i: (0, 0), memory_space=pltpu.SMEM),
        ],
        out_specs=pl.BlockSpec((VB,), lambda i: (i,)),
        out_shape=jax.ShapeDtypeStruct((PV,), jnp.float32),
    )(table_t, w_col, b2)


def _gather_body(p_hbm, idx_hbm, out_hbm, p_sh,
                 idx_v0, idx_v1, idx_v2, val_v0, val_v1,
                 sem_stage, si0, si1, si2, sg0, sg1, so0, so1):
    sid = lax.axis_index("s")
    wid = sid * NC + lax.axis_index("c")
    base = wid * PER_W
    idxs = (idx_v0, idx_v1, idx_v2)
    vals = (val_v0, val_v1)
    si = (si0, si1, si2)
    sg = (sg0, sg1)
    so = (so0, so1)

    def load_idx(j):
        return pltpu.async_copy(
            idx_hbm.at[pl.ds(base + j * CHUNK, CHUNK)], idxs[j % 3], si[j % 3])

    def gather(j):
        src = p_hbm if j % 5 == 2 else p_sh
        return pltpu.async_copy(src.at[idxs[j % 3]], vals[j % 2], sg[j % 2])

    def store(j):
        return pltpu.async_copy(
            vals[j % 2], out_hbm.at[pl.ds(base + j * CHUNK, CHUNK)], so[j % 2])

    h_stage = pltpu.async_copy(p_hbm.at[pl.ds(sid * SEG, SEG)],
                               p_sh.at[pl.ds(sid * SEG, SEG)], sem_stage)
    h_i = {j: load_idx(j) for j in range(min(3, N_CHUNKS))}
    h_stage.wait()
    plsc.subcore_barrier()
    h_i[0].wait()
    h_g = {0: gather(0)}
    h_o = {}
    for j in range(N_CHUNKS):
        if j + 1 < N_CHUNKS:
            h_i[j + 1].wait()
            if j >= 1:
                h_o[j - 1].wait()
            h_g[j + 1] = gather(j + 1)
        h_g[j].wait()
        if j + 3 < N_CHUNKS:
            h_i[j + 3] = load_idx(j + 3)
        h_o[j] = store(j)
    h_o[N_CHUNKS - 2].wait()
    h_o[N_CHUNKS - 1].wait()


_sc_gather = functools.partial(
    pl.kernel,
    mesh=plsc.VectorSubcoreMesh(core_axis_name="c", subcore_axis_name="s"),
    out_type=jax.ShapeDtypeStruct((N_TOK,), jnp.float32),
    scratch_types=[
        pltpu.VMEM_SHARED((PV,), jnp.float32),
        pltpu.VMEM((CHUNK,), jnp.int32),
        pltpu.VMEM((CHUNK,), jnp.int32),
        pltpu.VMEM((CHUNK,), jnp.int32),
        pltpu.VMEM((CHUNK,), jnp.float32),
        pltpu.VMEM((CHUNK,), jnp.float32),
        pltpu.SemaphoreType.DMA,
        pltpu.SemaphoreType.DMA,
        pltpu.SemaphoreType.DMA,
        pltpu.SemaphoreType.DMA,
        pltpu.SemaphoreType.DMA,
        pltpu.SemaphoreType.DMA,
        pltpu.SemaphoreType.DMA,
        pltpu.SemaphoreType.DMA,
    ],
)(_gather_body)


def kernel(token_ids, embed_table, W, b):
    table_t = embed_table.T
    w_col = W.reshape(DIM, 1)
    b2 = jnp.broadcast_to(b.astype(jnp.float32), (1, 1))
    p = _project(table_t, w_col, b2)
    idx = (token_ids.T.astype(jnp.int32)
           .reshape(L // 8, 8, B // 128, 128)
           .swapaxes(1, 2)
           .reshape(N_TOK))
    out = _sc_gather(p, idx)
    return (out.reshape(L // 8, B // 128, 8, 128)
            .swapaxes(1, 2)
            .reshape(L, B)
            .T)


# speedup vs baseline: 1.1457x; 1.1457x over previous
import functools

import jax
import jax.numpy as jnp
from jax import lax
from jax.experimental import pallas as pl
from jax.experimental.pallas import tpu as pltpu
from jax.experimental.pallas import tpu_sc as plsc

VOCAB = 1000000
DIM = 32
B = 16384
L = 200

VB = 32768
N_VBLK = 31
PV = N_VBLK * VB

NC, NS = 2, 16
NW = NC * NS
N_TOK = B * L
PER_W = N_TOK // NW
CHUNK = 4096
N_CHUNKS = PER_W // CHUNK
SEG = PV // NS


def _project_body(x_ref, w_ref, b_ref, o_ref):
    o_ref[:] = jnp.sum(x_ref[:] * w_ref[:], axis=0) + b_ref[0, 0]


def _project(table_t, w_col, b2):
    return pl.pallas_call(
        _project_body,
        grid=(N_VBLK,),
        in_specs=[
            pl.BlockSpec((DIM, VB), lambda i: (0, i)),
            pl.BlockSpec((DIM, 1), lambda i: (0, 0)),
            pl.BlockSpec((1, 1), lambda i: (0, 0), memory_space=pltpu.SMEM),
        ],
        out_specs=pl.BlockSpec((VB,), lambda i: (i,)),
        out_shape=jax.ShapeDtypeStruct((PV,), jnp.float32),
    )(table_t, w_col, b2)


def _gather_body(p_hbm, idx_hbm, out_hbm, p_sh,
                 idx_v0, idx_v1, idx_v2, val_v0, val_v1,
                 sem_stage, si0, si1, si2, sg0, sg1, so0, so1):
    sid = lax.axis_index("s")
    wid = sid * NC + lax.axis_index("c")
    base = wid * PER_W
    idxs = (idx_v0, idx_v1, idx_v2)
    vals = (val_v0, val_v1)
    si = (si0, si1, si2)
    sg = (sg0, sg1)
    so = (so0, so1)

    def load_idx(j):
        return pltpu.async_copy(
            idx_hbm.at[pl.ds(base + j * CHUNK, CHUNK)], idxs[j % 3], si[j % 3])

    def gather(j):
        return pltpu.async_copy(p_sh.at[idxs[j % 3]], vals[j % 2], sg[j % 2])

    def store(j):
        return pltpu.async_copy(
            vals[j % 2], out_hbm.at[pl.ds(base + j * CHUNK, CHUNK)], so[j % 2])

    h_stage = pltpu.async_copy(p_hbm.at[pl.ds(sid * SEG, SEG)],
                               p_sh.at[pl.ds(sid * SEG, SEG)], sem_stage)
    h_i = {j: load_idx(j) for j in range(min(3, N_CHUNKS))}
    h_stage.wait()
    plsc.subcore_barrier()
    h_i[0].wait()
    h_g = {0: gather(0)}
    h_o = {}
    for j in range(N_CHUNKS):
        if j + 1 < N_CHUNKS:
            h_i[j + 1].wait()
            if j >= 1:
                h_o[j - 1].wait()
            h_g[j + 1] = gather(j + 1)
        h_g[j].wait()
        if j + 3 < N_CHUNKS:
            h_i[j + 3] = load_idx(j + 3)
        h_o[j] = store(j)
    h_o[N_CHUNKS - 2].wait()
    h_o[N_CHUNKS - 1].wait()


_sc_gather = functools.partial(
    pl.kernel,
    mesh=plsc.VectorSubcoreMesh(core_axis_name="c", subcore_axis_name="s"),
    out_type=jax.ShapeDtypeStruct((N_TOK,), jnp.float32),
    scratch_types=[
        pltpu.VMEM_SHARED((PV,), jnp.float32),
        pltpu.VMEM((CHUNK,), jnp.int32),
        pltpu.VMEM((CHUNK,), jnp.int32),
        pltpu.VMEM((CHUNK,), jnp.int32),
        pltpu.VMEM((CHUNK,), jnp.float32),
        pltpu.VMEM((CHUNK,), jnp.float32),
        pltpu.SemaphoreType.DMA,
        pltpu.SemaphoreType.DMA,
        pltpu.SemaphoreType.DMA,
        pltpu.SemaphoreType.DMA,
        pltpu.SemaphoreType.DMA,
        pltpu.SemaphoreType.DMA,
        pltpu.SemaphoreType.DMA,
        pltpu.SemaphoreType.DMA,
    ],
)(_gather_body)


def kernel(token_ids, embed_table, W, b):
    table_t = embed_table.T
    w_col = W.reshape(DIM, 1)
    b2 = jnp.broadcast_to(b.astype(jnp.float32), (1, 1))
    p = _project(table_t, w_col, b2)
    idx = (token_ids.T.astype(jnp.int32)
           .reshape(L // 8, 8, B // 128, 128)
           .swapaxes(1, 2)
           .reshape(N_TOK))
    out = _sc_gather(p, idx)
    return (out.reshape(L // 8, B // 128, 8, 128)
            .swapaxes(1, 2)
            .reshape(L, B)
            .T)
